# baseline (speedup 1.0000x reference)
import jax
import jax.numpy as jnp
from jax import lax
from jax.experimental import pallas as pl
from jax.experimental.pallas import tpu as pltpu
from jax.experimental.pallas import tpu_sc as plsc

B = 16384
D = 128
S = 17
L = 16

NC = 2
NS = 16
NW = NC * NS

ROWS_W = B // NW
ROWS_C = 16
PAIRS_C = ROWS_C * S
CHUNKS_W = ROWS_W // ROWS_C
PAIRS_W = ROWS_W * S
NSEG = D // L
NBUF = 2


def _sc_body(idx_hbm, ctx_hbm, w_hbm, b_hbm, out_hbm,
             idx_v, bias_v, logits_v, acc_v, wbufs, ctxbufs,
             semw, semc, semb):
    wid = lax.axis_index("s") * NC + lax.axis_index("c")
    row0 = wid * ROWS_W

    pltpu.sync_copy(idx_hbm.at[pl.ds(wid * PAIRS_W, PAIRS_W)], idx_v)
    pltpu.async_copy(b_hbm.at[idx_v], bias_v, semb)

    def fire(c, t):
        pltpu.async_copy(w_hbm.at[idx_v.at[pl.ds(c * PAIRS_C, PAIRS_C)]],
                         wbufs[t], semw[t])
        pltpu.async_copy(ctx_hbm.at[pl.ds(row0 + c * ROWS_C, ROWS_C)],
                         ctxbufs[t], semc[t])

    def drain(t):
        pltpu.make_async_copy(w_hbm.at[pl.ds(0, PAIRS_C)], wbufs[t],
                              semw[t]).wait()
        pltpu.make_async_copy(ctx_hbm.at[pl.ds(0, ROWS_C)], ctxbufs[t],
                              semc[t]).wait()

    iota16 = lax.iota(jnp.int32, L)

    def compute(c, t):
        wbuf, ctxbuf = wbufs[t], ctxbufs[t]

        @plsc.parallel_loop(0, ROWS_C, 1, unroll=2)
        def i_body(i):
            ctx = [ctxbuf[i, pl.ds(k * L, L)] for k in range(NSEG)]
            for s in range(S):
                row = i * S + s
                a0 = wbuf[row, pl.ds(0, L)] * ctx[0]
                a1 = wbuf[row, pl.ds(L, L)] * ctx[1]
                for k in range(2, NSEG, 2):
                    a0 = a0 + wbuf[row, pl.ds(k * L, L)] * ctx[k]
                    a1 = a1 + wbuf[row, pl.ds((k + 1) * L, L)] * ctx[k + 1]
                acc_v[row, pl.ds(0, L)] = a0 + a1

        for grp in range((PAIRS_C + L - 1) // L):
            rows = grp * L + iota16
            tot = plsc.load_gather(acc_v, [rows, jnp.zeros((L,), jnp.int32)])
            for d in range(1, L):
                tot = tot + plsc.load_gather(
                    acc_v, [rows, jnp.full((L,), d, jnp.int32)])
            pair = c * PAIRS_C + grp * L + iota16
            mask = None
            if grp * L + L > PAIRS_C:
                mask = iota16 < (PAIRS_C - grp * L)
            plsc.store_scatter(logits_v, [pair // S, pair % S], tot,
                               mask=mask)

    for t in range(NBUF):
        fire(t, t)

    def loop_body(g, _):
        for t in range(NBUF):
            c = g * NBUF + t
            drain(t)
            compute(c, t)

            @pl.when(c + NBUF < CHUNKS_W)
            def _(c=c, t=t):
                fire(c + NBUF, t)

        return 0

    lax.fori_loop(0, CHUNKS_W // NBUF, loop_body, 0)

    pltpu.make_async_copy(b_hbm.at[pl.ds(0, PAIRS_W)], bias_v, semb).wait()

    iota = lax.iota(jnp.int32, L)

    def jblock(j, _):
        b = j * L + iota
        base = b * S
        xs = []
        for s in range(S):
            lg = plsc.load_gather(logits_v, [b, jnp.full((L,), s, jnp.int32)])
            bi = plsc.load_gather(bias_v, [base + s])
            xs.append(lg + bi)
        m = xs[0]
        for s in range(1, S):
            m = jnp.maximum(m, xs[s])
        es = [jnp.exp(x - m) for x in xs]
        tot = es[0]
        for s in range(1, S):
            tot = tot + es[s]
        r = 1.0 / tot
        for s in range(S):
            plsc.store_scatter(logits_v, [b, jnp.full((L,), s, jnp.int32)],
                               es[s] * r)
        return 0

    lax.fori_loop(0, ROWS_W // L, jblock, 0)

    pltpu.sync_copy(logits_v, out_hbm.at[pl.ds(row0, ROWS_W)])


@jax.jit
def _run(idx, context, w_table, b_flat):
    mesh = plsc.VectorSubcoreMesh(core_axis_name="c", subcore_axis_name="s",
                                  num_cores=NC, num_subcores=NS)
    return pl.kernel(
        _sc_body,
        out_type=jax.ShapeDtypeStruct((B, S), jnp.float32),
        mesh=mesh,
        compiler_params=pltpu.CompilerParams(needs_layout_passes=False,
                                             use_tc_tiling_on_sc=False),
        scratch_types=[
            pltpu.VMEM((PAIRS_W,), jnp.int32),
            pltpu.VMEM((PAIRS_W,), jnp.float32),
            pltpu.VMEM((ROWS_W, S), jnp.float32),
            pltpu.VMEM((((PAIRS_C + L - 1) // L) * L, L), jnp.float32),
            [pltpu.VMEM((PAIRS_C, D), jnp.float32) for _ in range(NBUF)],
            [pltpu.VMEM((ROWS_C, D), jnp.float32) for _ in range(NBUF)],
            [pltpu.SemaphoreType.DMA for _ in range(NBUF)],
            [pltpu.SemaphoreType.DMA for _ in range(NBUF)],
            pltpu.SemaphoreType.DMA,
        ],
    )(idx, context, w_table, b_flat)


def kernel(samples, context, w_table, b_table):
    idx = samples.astype(jnp.int32).reshape(-1)
    return _run(idx, context, w_table, b_table.reshape(-1))


# speedup vs baseline: 1.0224x; 1.0224x over previous
import jax
import jax.numpy as jnp
from jax import lax
from jax.experimental import pallas as pl
from jax.experimental.pallas import tpu as pltpu
from jax.experimental.pallas import tpu_sc as plsc

B = 16384
D = 128
S = 17
L = 16

NC = 2
NS = 16
NW = NC * NS

ROWS_W = B // NW
ROWS_C = 8
PAIRS_C = ROWS_C * S
CHUNKS_W = ROWS_W // ROWS_C
PAIRS_W = ROWS_W * S
NSEG = D // L
NBUF = 2


def _sc_body(idx_hbm, ctx_hbm, w_hbm, b_hbm, out_hbm,
             idx_v, bias_v, logits_v, acc_v, wbufs, ctxbufs,
             semw, semc, semb):
    wid = lax.axis_index("s") * NC + lax.axis_index("c")
    row0 = wid * ROWS_W

    pltpu.sync_copy(idx_hbm.at[pl.ds(wid * PAIRS_W, PAIRS_W)], idx_v)
    pltpu.async_copy(b_hbm.at[idx_v], bias_v, semb)

    def fire(c, t):
        pltpu.async_copy(w_hbm.at[idx_v.at[pl.ds(c * PAIRS_C, PAIRS_C)]],
                         wbufs[t], semw[t])
        pltpu.async_copy(ctx_hbm.at[pl.ds(row0 + c * ROWS_C, ROWS_C)],
                         ctxbufs[t], semc[t])

    def drain(t):
        pltpu.make_async_copy(w_hbm.at[pl.ds(0, PAIRS_C)], wbufs[t],
                              semw[t]).wait()
        pltpu.make_async_copy(ctx_hbm.at[pl.ds(0, ROWS_C)], ctxbufs[t],
                              semc[t]).wait()

    iota16 = lax.iota(jnp.int32, L)

    def compute(c, t):
        wbuf, ctxbuf = wbufs[t], ctxbufs[t]

        @plsc.parallel_loop(0, ROWS_C, 1, unroll=1)
        def i_body(i):
            ctx = [ctxbuf[i, pl.ds(k * L, L)] for k in range(NSEG)]

            @plsc.parallel_loop(0, S, 1, unroll=2)
            def s_body(s):
                row = i * S + s
                a0 = wbuf[row, pl.ds(0, L)] * ctx[0]
                a1 = wbuf[row, pl.ds(L, L)] * ctx[1]
                for k in range(2, NSEG, 2):
                    a0 = a0 + wbuf[row, pl.ds(k * L, L)] * ctx[k]
                    a1 = a1 + wbuf[row, pl.ds((k + 1) * L, L)] * ctx[k + 1]
                acc_v[row, pl.ds(0, L)] = a0 + a1

        for grp in range((PAIRS_C + L - 1) // L):
            rows = grp * L + iota16
            tot = plsc.load_gather(acc_v, [rows, jnp.zeros((L,), jnp.int32)])
            for d in range(1, L):
                tot = tot + plsc.load_gather(
                    acc_v, [rows, jnp.full((L,), d, jnp.int32)])
            pair = c * PAIRS_C + grp * L + iota16
            mask = None
            if grp * L + L > PAIRS_C:
                mask = iota16 < (PAIRS_C - grp * L)
            plsc.store_scatter(logits_v, [pair // S, pair % S], tot,
                               mask=mask)

    for t in range(NBUF):
        fire(t, t)

    def loop_body(g, _):
        for t in range(NBUF):
            c = g * NBUF + t
            drain(t)
            compute(c, t)

            @pl.when(c + NBUF < CHUNKS_W)
            def _(c=c, t=t):
                fire(c + NBUF, t)

        return 0

    lax.fori_loop(0, CHUNKS_W // NBUF, loop_body, 0)

    pltpu.make_async_copy(b_hbm.at[pl.ds(0, PAIRS_W)], bias_v, semb).wait()

    iota = lax.iota(jnp.int32, L)

    def jblock(j, _):
        b = j * L + iota
        base = b * S
        xs = []
        for s in range(S):
            lg = plsc.load_gather(logits_v, [b, jnp.full((L,), s, jnp.int32)])
            bi = plsc.load_gather(bias_v, [base + s])
            xs.append(lg + bi)
        m = xs[0]
        for s in range(1, S):
            m = jnp.maximum(m, xs[s])
        es = [jnp.exp(x - m) for x in xs]
        tot = es[0]
        for s in range(1, S):
            tot = tot + es[s]
        r = 1.0 / tot
        for s in range(S):
            plsc.store_scatter(logits_v, [b, jnp.full((L,), s, jnp.int32)],
                               es[s] * r)
        return 0

    lax.fori_loop(0, ROWS_W // L, jblock, 0)

    pltpu.sync_copy(logits_v, out_hbm.at[pl.ds(row0, ROWS_W)])


@jax.jit
def _run(idx, context, w_table, b_flat):
    mesh = plsc.VectorSubcoreMesh(core_axis_name="c", subcore_axis_name="s",
                                  num_cores=NC, num_subcores=NS)
    return pl.kernel(
        _sc_body,
        out_type=jax.ShapeDtypeStruct((B, S), jnp.float32),
        mesh=mesh,
        compiler_params=pltpu.CompilerParams(needs_layout_passes=False,
                                             use_tc_tiling_on_sc=False),
        scratch_types=[
            pltpu.VMEM((PAIRS_W,), jnp.int32),
            pltpu.VMEM((PAIRS_W,), jnp.float32),
            pltpu.VMEM((ROWS_W, S), jnp.float32),
            pltpu.VMEM((((PAIRS_C + L - 1) // L) * L, L), jnp.float32),
            [pltpu.VMEM((PAIRS_C, D), jnp.float32) for _ in range(NBUF)],
            [pltpu.VMEM((ROWS_C, D), jnp.float32) for _ in range(NBUF)],
            [pltpu.SemaphoreType.DMA for _ in range(NBUF)],
            [pltpu.SemaphoreType.DMA for _ in range(NBUF)],
            pltpu.SemaphoreType.DMA,
        ],
    )(idx, context, w_table, b_flat)


def kernel(samples, context, w_table, b_table):
    idx = samples.astype(jnp.int32).reshape(-1)
    return _run(idx, context, w_table, b_table.reshape(-1))


# speedup vs baseline: 1.0512x; 1.0281x over previous
import jax
import jax.numpy as jnp
from jax import lax
from jax.experimental import pallas as pl
from jax.experimental.pallas import tpu as pltpu
from jax.experimental.pallas import tpu_sc as plsc

B = 16384
D = 128
S = 17
L = 16

NC = 2
NS = 16
NW = NC * NS

ROWS_W = B // NW
ROWS_C = 8
PAIRS_C = ROWS_C * S
CHUNKS_W = ROWS_W // ROWS_C
PAIRS_W = ROWS_W * S
NSEG = D // L
NBUF = 2


def _sc_body(idx_hbm, ctx_hbm, w_hbm, b_hbm, out_hbm,
             idx_v, bias_v, logits_v, acc_v, wbufs, ctxbufs,
             semw, semc, semb):
    wid = lax.axis_index("s") * NC + lax.axis_index("c")
    row0 = wid * ROWS_W

    pltpu.sync_copy(idx_hbm.at[pl.ds(wid * PAIRS_W, PAIRS_W)], idx_v)
    pltpu.async_copy(b_hbm.at[idx_v], bias_v, semb)

    def fire(c, t):
        pltpu.async_copy(w_hbm.at[idx_v.at[pl.ds(c * PAIRS_C, PAIRS_C)]],
                         wbufs[t], semw[t])
        pltpu.async_copy(ctx_hbm.at[pl.ds(row0 + c * ROWS_C, ROWS_C)],
                         ctxbufs[t], semc[t])

    def drain(t):
        pltpu.make_async_copy(w_hbm.at[pl.ds(0, PAIRS_C)], wbufs[t],
                              semw[t]).wait()
        pltpu.make_async_copy(ctx_hbm.at[pl.ds(0, ROWS_C)], ctxbufs[t],
                              semc[t]).wait()

    iota16 = lax.iota(jnp.int32, L)

    def compute(c, t):
        wbuf, ctxbuf = wbufs[t], ctxbufs[t]

        @plsc.parallel_loop(0, ROWS_C, 1, unroll=1)
        def i_body(i):
            ctx = [ctxbuf[i, pl.ds(k * L, L)] for k in range(NSEG)]

            @plsc.parallel_loop(0, S, 1, unroll=2)
            def s_body(s):
                row = i * S + s
                a = [wbuf[row, pl.ds(k * L, L)] * ctx[k] for k in range(4)]
                for k in range(4, NSEG):
                    a[k % 4] = a[k % 4] + wbuf[row, pl.ds(k * L, L)] * ctx[k]
                acc_v[row, pl.ds(0, L)] = (a[0] + a[1]) + (a[2] + a[3])

        nfull = PAIRS_C // L

        @plsc.parallel_loop(0, nfull, 1, unroll=1)
        def g_body(grp):
            rows = grp * L + iota16
            tot = plsc.load_gather(acc_v, [rows, jnp.zeros((L,), jnp.int32)])
            for d in range(1, L):
                tot = tot + plsc.load_gather(
                    acc_v, [rows, jnp.full((L,), d, jnp.int32)])
            pair = c * PAIRS_C + grp * L + iota16
            plsc.store_scatter(logits_v, [pair // S, pair % S], tot)

        if PAIRS_C % L:
            grp = nfull
            rows = grp * L + iota16
            tot = plsc.load_gather(acc_v, [rows, jnp.zeros((L,), jnp.int32)])
            for d in range(1, L):
                tot = tot + plsc.load_gather(
                    acc_v, [rows, jnp.full((L,), d, jnp.int32)])
            pair = c * PAIRS_C + grp * L + iota16
            plsc.store_scatter(logits_v, [pair // S, pair % S], tot,
                               mask=iota16 < (PAIRS_C - grp * L))

    for t in range(NBUF):
        fire(t, t)

    def loop_body(g, _):
        for t in range(NBUF):
            c = g * NBUF + t
            drain(t)
            compute(c, t)

            @pl.when(c + NBUF < CHUNKS_W)
            def _(c=c, t=t):
                fire(c + NBUF, t)

        return 0

    lax.fori_loop(0, CHUNKS_W // NBUF, loop_body, 0)

    pltpu.make_async_copy(b_hbm.at[pl.ds(0, PAIRS_W)], bias_v, semb).wait()

    iota = lax.iota(jnp.int32, L)

    def jblock(j, _):
        b = j * L + iota
        base = b * S
        xs = []
        for s in range(S):
            lg = plsc.load_gather(logits_v, [b, jnp.full((L,), s, jnp.int32)])
            bi = plsc.load_gather(bias_v, [base + s])
            xs.append(lg + bi)
        m = xs[0]
        for s in range(1, S):
            m = jnp.maximum(m, xs[s])
        es = [jnp.exp(x - m) for x in xs]
        tot = es[0]
        for s in range(1, S):
            tot = tot + es[s]
        r = 1.0 / tot
        for s in range(S):
            plsc.store_scatter(logits_v, [b, jnp.full((L,), s, jnp.int32)],
                               es[s] * r)
        return 0

    lax.fori_loop(0, ROWS_W // L, jblock, 0)

    pltpu.sync_copy(logits_v, out_hbm.at[pl.ds(row0, ROWS_W)])


@jax.jit
def _run(idx, context, w_table, b_flat):
    mesh = plsc.VectorSubcoreMesh(core_axis_name="c", subcore_axis_name="s",
                                  num_cores=NC, num_subcores=NS)
    return pl.kernel(
        _sc_body,
        out_type=jax.ShapeDtypeStruct((B, S), jnp.float32),
        mesh=mesh,
        compiler_params=pltpu.CompilerParams(needs_layout_passes=False,
                                             use_tc_tiling_on_sc=False),
        scratch_types=[
            pltpu.VMEM((PAIRS_W,), jnp.int32),
            pltpu.VMEM((PAIRS_W,), jnp.float32),
            pltpu.VMEM((ROWS_W, S), jnp.float32),
            pltpu.VMEM((((PAIRS_C + L - 1) // L) * L, L), jnp.float32),
            [pltpu.VMEM((PAIRS_C, D), jnp.float32) for _ in range(NBUF)],
            [pltpu.VMEM((ROWS_C, D), jnp.float32) for _ in range(NBUF)],
            [pltpu.SemaphoreType.DMA for _ in range(NBUF)],
            [pltpu.SemaphoreType.DMA for _ in range(NBUF)],
            pltpu.SemaphoreType.DMA,
        ],
    )(idx, context, w_table, b_flat)


def kernel(samples, context, w_table, b_table):
    idx = samples.astype(jnp.int32).reshape(-1)
    return _run(idx, context, w_table, b_table.reshape(-1))
